# Initial kernel scaffold; baseline (speedup 1.0000x reference)
import jax
import jax.numpy as jnp
from jax.experimental import pallas as pl


def kernel(node_states, edges, kernel):
    raise NotImplementedError("write your pallas kernel here")


# speedup vs baseline: 1.7790x; 1.7790x over previous
import jax
import jax.numpy as jnp
from jax import lax
from jax.experimental import pallas as pl
from jax.experimental.pallas import tpu as pltpu
from jax.experimental.pallas import tpu_sc as plsc

N_NODES = 10000
N_EDGES = 160000
D = 256
NSL = D // 16
W_SC = 384
PAD_NODE = 10016

NC = 2
NS = 16
L = 16
NW = NC * NS

EP = 163840
E_PER_TILE = EP // NW
CH = 64
NCHUNK = E_PER_TILE // CH

NB = 11264
TAIL_ROW = 10240
DUMMY_ROW = 10272
ZROWS_PER_SUB = NB // NS


def _tc1_body(x_ref, w_ref, t_ref):
    t_ref[...] = jnp.dot(x_ref[...], w_ref[...],
                         preferred_element_type=jnp.float32)


def _tc_transform(x, w):
    blk = 1000
    return pl.pallas_call(
        _tc1_body,
        grid=(N_NODES // blk,),
        in_specs=[
            pl.BlockSpec((blk, D), lambda i: (i, 0)),
            pl.BlockSpec((D, D), lambda i: (0, 0)),
        ],
        out_specs=pl.BlockSpec((blk, D), lambda i: (i, 0)),
        out_shape=jax.ShapeDtypeStruct((N_NODES, D), jnp.float32),
    )(x, w)


def _hsum16(v, lanes):
    for k in (8, 4, 2, 1):
        v = v + v.at[lanes ^ k].get(mode="promise_in_bounds")
    return v


def _rsqrt_newton(x):
    i = lax.bitcast_convert_type(x, jnp.int32)
    i = jnp.int32(0x5F3759DF) - lax.shift_right_logical(i, 1)
    y = lax.bitcast_convert_type(i, jnp.float32)
    for _ in range(3):
        y = y * (1.5 - 0.5 * x * y * y)
    return y


def _sc_body(t_hbm, dst_hbm, dnx_hbm, src_hbm, num_hbm,
             dst_v, dnx_v, src_v, iv2, t_v, s_v, w_v, acc_v, sem):
    c = lax.axis_index("c")
    s = lax.axis_index("s")
    wid = c * NS + s
    ebase = wid * E_PER_TILE
    rofs = c * NB
    lanes = lax.iota(jnp.int32, L)
    zero16 = jnp.zeros((L,), jnp.float32)
    z16i = jnp.zeros((L,), jnp.int32)
    dummy_sp = z16i + (rofs + DUMMY_ROW + wid)

    def _zero_row(r, carry):
        for j in range(W_SC // L):
            w_v[r, pl.ds(j * L, L)] = zero16
        return carry

    lax.fori_loop(0, CH, _zero_row, None)
    for j in range(W_SC // L):
        acc_v[0, pl.ds(j * L, L)] = zero16
    for k in range(ZROWS_PER_SUB // CH):
        pltpu.sync_copy(
            w_v, num_hbm.at[pl.ds(rofs + s * ZROWS_PER_SUB + k * CH, CH)])
    plsc.subcore_barrier()

    def _chunk(i, carry):
        prev_sp, open_f = carry
        cb = ebase + i * CH
        pltpu.sync_copy(dst_hbm.at[pl.ds(cb, CH)], dst_v)
        pltpu.sync_copy(dnx_hbm.at[pl.ds(cb, CH)], dnx_v)
        pltpu.sync_copy(src_hbm.at[pl.ds(cb, CH)], src_v)
        pltpu.async_copy(t_hbm.at[dst_v], t_v, sem).wait()
        pltpu.async_copy(t_hbm.at[src_v], s_v, sem).wait()

        def _group(g, carry2):
            prev2, open2 = carry2
            gb = pl.multiple_of(g * L, L)
            d16 = dst_v[pl.ds(gb, L)]
            dn16 = dnx_v[pl.ds(gb, L)]

            def _edge(e0, carry3):
                prev3, open3, ivcol = carry3
                e = g * L + e0
                acc_ts = zero16
                acc_tt = zero16
                acc_ss = zero16
                for j in range(NSL):
                    sl = pl.ds(j * L, L)
                    tj = t_v[e, sl]
                    sj = s_v[e, sl]
                    acc_ts = acc_ts + tj * sj
                    acc_tt = acc_tt + tj * tj
                    acc_ss = acc_ss + sj * sj
                acc_ts = _hsum16(acc_ts, lanes)
                acc_tt = _hsum16(acc_tt, lanes)
                acc_ss = _hsum16(acc_ss, lanes)
                x = jnp.minimum(
                    jnp.maximum(acc_tt, 1e-8) * jnp.maximum(acc_ss, 1e-8),
                    1e30)
                cos = acc_ts * _rsqrt_newton(x)
                pv = jnp.exp(cos - 1.0)

                e016 = z16i + e0
                d_sp = d16.at[e016].get(mode="promise_in_bounds")
                dn_sp = dn16.at[e016].get(mode="promise_in_bounds")
                same = jnp.where(d_sp == prev3, 1.0, 0.0)
                for j in range(NSL):
                    sl = pl.ds(j * L, L)
                    a = acc_v[0, sl] * same + s_v[e, sl] * pv
                    acc_v[0, sl] = a
                    w_v[e, sl] = a
                slq = pl.ds(D, L)
                aq = acc_v[0, slq] * same + jnp.where(lanes == 0, pv, 0.0)
                acc_v[0, slq] = aq
                w_v[e, slq] = aq
                is_last = d_sp != dn_sp
                ivcol = jnp.where(lanes == e0,
                                  jnp.where(is_last, d_sp + rofs, dummy_sp),
                                  ivcol)
                open3 = jnp.where(is_last, 0.0, 1.0)
                return (d_sp, open3, ivcol)

            prev2, open2, ivcol = lax.fori_loop(
                0, L, _edge, (prev2, open2, z16i))
            iv2[pl.ds(gb, L)] = ivcol
            return (prev2, open2)

        carry = lax.fori_loop(0, CH // L, _group, (prev_sp, open_f))
        pltpu.async_copy(w_v, num_hbm.at[iv2], sem).wait()
        return carry

    init = (z16i - 1, zero16)
    prev_sp, open_f = lax.fori_loop(0, NCHUNK, _chunk, init)

    for j in range(NSL + 1):
        sl = pl.ds(j * L, L)
        w_v[0, sl] = acc_v[0, sl] * open_f
    tail_sp = z16i + (rofs + TAIL_ROW + wid)
    tgt = jnp.where(open_f > 0.5, tail_sp, dummy_sp)
    iv2[pl.ds(0, L)] = jnp.where(lanes == 0, tgt, dummy_sp)
    for g in range(1, CH // L):
        iv2[pl.ds(g * L, L)] = dummy_sp
    pltpu.async_copy(w_v, num_hbm.at[iv2], sem).wait()


def _sc_pass(t, dstp, dnx, srcp):
    mesh = plsc.VectorSubcoreMesh(
        core_axis_name="c", subcore_axis_name="s",
        num_cores=NC, num_subcores=NS)
    kfn = pl.kernel(
        _sc_body,
        out_type=[jax.ShapeDtypeStruct((NC * NB, W_SC), jnp.float32)],
        mesh=mesh,
        scratch_types=[
            pltpu.VMEM((CH,), jnp.int32),
            pltpu.VMEM((CH,), jnp.int32),
            pltpu.VMEM((CH,), jnp.int32),
            pltpu.VMEM((CH,), jnp.int32),
            pltpu.VMEM((CH, D), jnp.float32),
            pltpu.VMEM((CH, D), jnp.float32),
            pltpu.VMEM((CH, W_SC), jnp.float32),
            pltpu.VMEM((1, W_SC), jnp.float32),
            pltpu.SemaphoreType.DMA,
        ],
    )
    return kfn(t, dstp, dnx, srcp)


def _tc2_body(n0_ref, n1_ref, t0_ref, t1_ref, dl_ref, out_ref):
    i = pl.program_id(0)
    blk = out_ref.shape[0]
    total = n0_ref[...] + n1_ref[...]
    tails = t0_ref[...] + t1_ref[...]
    rows = jax.lax.broadcasted_iota(jnp.int32, (blk, NW), 0) + i * blk
    h = jnp.where(rows == dl_ref[...], 1.0, 0.0)
    total = total + jnp.dot(h, tails, preferred_element_type=jnp.float32)
    q = total[:, D]
    inv = 1.0 / (q + 1e-8)
    out_ref[...] = total[:, :D] * inv[:, None]


def _tc_finalize(num, dl):
    blk = 1000
    num2 = num.reshape(NC, NB, W_SC)
    tb = TAIL_ROW // 32
    return pl.pallas_call(
        _tc2_body,
        grid=(N_NODES // blk,),
        in_specs=[
            pl.BlockSpec((blk, W_SC), lambda i: (i, 0)),
            pl.BlockSpec((blk, W_SC), lambda i: (i, 0)),
            pl.BlockSpec((32, W_SC), lambda i: (tb, 0)),
            pl.BlockSpec((32, W_SC), lambda i: (tb, 0)),
            pl.BlockSpec((1, NW), lambda i: (0, 0)),
        ],
        out_specs=pl.BlockSpec((blk, D), lambda i: (i, 0)),
        out_shape=jax.ShapeDtypeStruct((N_NODES, D), jnp.float32),
    )(num2[0], num2[1], num2[0], num2[1], dl.reshape(1, NW))


def kernel(node_states, edges, kernel):
    w = kernel
    dst = jnp.asarray(edges[:, 0], jnp.int32)
    src = jnp.asarray(edges[:, 1], jnp.int32)
    ds_, ss_ = lax.sort((dst, src), num_keys=1)
    npad = EP - N_EDGES
    dstp = jnp.concatenate([ds_, jnp.full((npad,), PAD_NODE, jnp.int32)])
    srcp = jnp.concatenate([ss_, jnp.zeros((npad,), jnp.int32)])
    dnx = jnp.concatenate([dstp[1:], jnp.full((1,), PAD_NODE + 1, jnp.int32)])
    dl = dstp[E_PER_TILE - 1::E_PER_TILE]
    t = _tc_transform(node_states, w)
    (num,) = _sc_pass(t, dstp, dnx, srcp)
    return _tc_finalize(num, dl)


# speedup vs baseline: 1.9023x; 1.0693x over previous
import jax
import jax.numpy as jnp
from jax import lax
from jax.experimental import pallas as pl
from jax.experimental.pallas import tpu as pltpu
from jax.experimental.pallas import tpu_sc as plsc

N_NODES = 10000
N_EDGES = 160000
D = 256
NSL = D // 16
W_SC = 384
PAD_NODE = 10016

NC = 2
NS = 16
L = 16
NW = NC * NS

EP = 163840
E_PER_TILE = EP // NW
CH = 64
NCHUNK = E_PER_TILE // CH

NB = 11264
TAIL_ROW = 10240
DUMMY_ROW = 10272
ZROWS_PER_SUB = NB // NS


def _tc1_body(x_ref, w_ref, t_ref):
    t_ref[...] = jnp.dot(x_ref[...], w_ref[...],
                         preferred_element_type=jnp.float32)


def _tc_transform(x, w):
    blk = 1000
    return pl.pallas_call(
        _tc1_body,
        grid=(N_NODES // blk,),
        in_specs=[
            pl.BlockSpec((blk, D), lambda i: (i, 0)),
            pl.BlockSpec((D, D), lambda i: (0, 0)),
        ],
        out_specs=pl.BlockSpec((blk, D), lambda i: (i, 0)),
        out_shape=jax.ShapeDtypeStruct((N_NODES, D), jnp.float32),
    )(x, w)


def _hsum16(v, lanes):
    for k in (8, 4, 2, 1):
        v = v + v.at[lanes ^ k].get(mode="promise_in_bounds")
    return v


def _rsqrt_newton(x):
    i = lax.bitcast_convert_type(x, jnp.int32)
    i = jnp.int32(0x5F3759DF) - lax.shift_right_logical(i, 1)
    y = lax.bitcast_convert_type(i, jnp.float32)
    for _ in range(3):
        y = y * (1.5 - 0.5 * x * y * y)
    return y


def _sc_body(t_hbm, dst_hbm, dnx_hbm, src_hbm, num_hbm,
             dst_v, dnx_v, src_v, iv2, t_v, s_v, w_v, acc_v, sem):
    c = lax.axis_index("c")
    s = lax.axis_index("s")
    wid = c * NS + s
    ebase = wid * E_PER_TILE
    rofs = c * NB
    lanes = lax.iota(jnp.int32, L)
    zero16 = jnp.zeros((L,), jnp.float32)
    z16i = jnp.zeros((L,), jnp.int32)
    dummy_sp = z16i + (rofs + DUMMY_ROW + wid)

    def _zero_row(r, carry):
        for j in range(W_SC // L):
            w_v[r, pl.ds(j * L, L)] = zero16
        return carry

    lax.fori_loop(0, CH, _zero_row, None)
    for j in range(W_SC // L):
        acc_v[0, pl.ds(j * L, L)] = zero16
    for k in range(ZROWS_PER_SUB // CH):
        pltpu.sync_copy(
            w_v, num_hbm.at[pl.ds(rofs + s * ZROWS_PER_SUB + k * CH, CH)])
    plsc.subcore_barrier()

    def _chunk(i, carry):
        prev_sp, open_f = carry
        cb = ebase + i * CH
        pltpu.sync_copy(dst_hbm.at[pl.ds(cb, CH)], dst_v)
        pltpu.sync_copy(dnx_hbm.at[pl.ds(cb, CH)], dnx_v)
        pltpu.sync_copy(src_hbm.at[pl.ds(cb, CH)], src_v)
        pltpu.async_copy(t_hbm.at[dst_v], t_v, sem).wait()
        pltpu.async_copy(t_hbm.at[src_v], s_v, sem).wait()

        def _group(g, carry2):
            prev2, open2 = carry2
            gb = pl.multiple_of(g * L, L)
            d16 = dst_v[pl.ds(gb, L)]
            dn16 = dnx_v[pl.ds(gb, L)]

            UNROLL = 4

            def _edge(eu, carry3):
                prev3, open3, ivcol = carry3
                pvs = []
                dsps = []
                dnsps = []
                for u in range(UNROLL):
                    e0 = eu * UNROLL + u
                    e = g * L + e0
                    acc_ts = zero16
                    acc_tt = zero16
                    acc_ss = zero16
                    for j in range(NSL):
                        sl = pl.ds(j * L, L)
                        tj = t_v[e, sl]
                        sj = s_v[e, sl]
                        acc_ts = acc_ts + tj * sj
                        acc_tt = acc_tt + tj * tj
                        acc_ss = acc_ss + sj * sj
                    acc_ts = _hsum16(acc_ts, lanes)
                    acc_tt = _hsum16(acc_tt, lanes)
                    acc_ss = _hsum16(acc_ss, lanes)
                    x = jnp.minimum(
                        jnp.maximum(acc_tt, 1e-8) * jnp.maximum(acc_ss, 1e-8),
                        1e30)
                    cos = acc_ts * _rsqrt_newton(x)
                    pvs.append(jnp.exp(cos - 1.0))
                    e016 = z16i + e0
                    dsps.append(d16.at[e016].get(mode="promise_in_bounds"))
                    dnsps.append(dn16.at[e016].get(mode="promise_in_bounds"))
                for u in range(UNROLL):
                    e0 = eu * UNROLL + u
                    e = g * L + e0
                    pv = pvs[u]
                    d_sp = dsps[u]
                    same = jnp.where(d_sp == prev3, 1.0, 0.0)
                    for j in range(NSL):
                        sl = pl.ds(j * L, L)
                        a = acc_v[0, sl] * same + s_v[e, sl] * pv
                        acc_v[0, sl] = a
                        w_v[e, sl] = a
                    slq = pl.ds(D, L)
                    aq = acc_v[0, slq] * same + jnp.where(lanes == 0, pv, 0.0)
                    acc_v[0, slq] = aq
                    w_v[e, slq] = aq
                    is_last = d_sp != dnsps[u]
                    ivcol = jnp.where(lanes == e0,
                                      jnp.where(is_last, d_sp + rofs,
                                                dummy_sp),
                                      ivcol)
                    open3 = jnp.where(is_last, 0.0, 1.0)
                    prev3 = d_sp
                return (prev3, open3, ivcol)

            prev2, open2, ivcol = lax.fori_loop(
                0, L // UNROLL, _edge, (prev2, open2, z16i))
            iv2[pl.ds(gb, L)] = ivcol
            return (prev2, open2)

        carry = lax.fori_loop(0, CH // L, _group, (prev_sp, open_f))
        pltpu.async_copy(w_v, num_hbm.at[iv2], sem).wait()
        return carry

    init = (z16i - 1, zero16)
    prev_sp, open_f = lax.fori_loop(0, NCHUNK, _chunk, init)

    for j in range(NSL + 1):
        sl = pl.ds(j * L, L)
        w_v[0, sl] = acc_v[0, sl] * open_f
    tail_sp = z16i + (rofs + TAIL_ROW + wid)
    tgt = jnp.where(open_f > 0.5, tail_sp, dummy_sp)
    iv2[pl.ds(0, L)] = jnp.where(lanes == 0, tgt, dummy_sp)
    for g in range(1, CH // L):
        iv2[pl.ds(g * L, L)] = dummy_sp
    pltpu.async_copy(w_v, num_hbm.at[iv2], sem).wait()


def _sc_pass(t, dstp, dnx, srcp):
    mesh = plsc.VectorSubcoreMesh(
        core_axis_name="c", subcore_axis_name="s",
        num_cores=NC, num_subcores=NS)
    kfn = pl.kernel(
        _sc_body,
        out_type=[jax.ShapeDtypeStruct((NC * NB, W_SC), jnp.float32)],
        mesh=mesh,
        scratch_types=[
            pltpu.VMEM((CH,), jnp.int32),
            pltpu.VMEM((CH,), jnp.int32),
            pltpu.VMEM((CH,), jnp.int32),
            pltpu.VMEM((CH,), jnp.int32),
            pltpu.VMEM((CH, D), jnp.float32),
            pltpu.VMEM((CH, D), jnp.float32),
            pltpu.VMEM((CH, W_SC), jnp.float32),
            pltpu.VMEM((1, W_SC), jnp.float32),
            pltpu.SemaphoreType.DMA,
        ],
    )
    return kfn(t, dstp, dnx, srcp)


def _tc2_body(n0_ref, n1_ref, t0_ref, t1_ref, dl_ref, out_ref):
    i = pl.program_id(0)
    blk = out_ref.shape[0]
    total = n0_ref[...] + n1_ref[...]
    tails = t0_ref[...] + t1_ref[...]
    rows = jax.lax.broadcasted_iota(jnp.int32, (blk, NW), 0) + i * blk
    h = jnp.where(rows == dl_ref[...], 1.0, 0.0)
    total = total + jnp.dot(h, tails, preferred_element_type=jnp.float32)
    q = total[:, D]
    inv = 1.0 / (q + 1e-8)
    out_ref[...] = total[:, :D] * inv[:, None]


def _tc_finalize(num, dl):
    blk = 1000
    num2 = num.reshape(NC, NB, W_SC)
    tb = TAIL_ROW // 32
    return pl.pallas_call(
        _tc2_body,
        grid=(N_NODES // blk,),
        in_specs=[
            pl.BlockSpec((blk, W_SC), lambda i: (i, 0)),
            pl.BlockSpec((blk, W_SC), lambda i: (i, 0)),
            pl.BlockSpec((32, W_SC), lambda i: (tb, 0)),
            pl.BlockSpec((32, W_SC), lambda i: (tb, 0)),
            pl.BlockSpec((1, NW), lambda i: (0, 0)),
        ],
        out_specs=pl.BlockSpec((blk, D), lambda i: (i, 0)),
        out_shape=jax.ShapeDtypeStruct((N_NODES, D), jnp.float32),
    )(num2[0], num2[1], num2[0], num2[1], dl.reshape(1, NW))


def kernel(node_states, edges, kernel):
    w = kernel
    dst = jnp.asarray(edges[:, 0], jnp.int32)
    src = jnp.asarray(edges[:, 1], jnp.int32)
    ds_, ss_ = lax.sort((dst, src), num_keys=1)
    npad = EP - N_EDGES
    dstp = jnp.concatenate([ds_, jnp.full((npad,), PAD_NODE, jnp.int32)])
    srcp = jnp.concatenate([ss_, jnp.zeros((npad,), jnp.int32)])
    dnx = jnp.concatenate([dstp[1:], jnp.full((1,), PAD_NODE + 1, jnp.int32)])
    dl = dstp[E_PER_TILE - 1::E_PER_TILE]
    t = _tc_transform(node_states, w)
    (num,) = _sc_pass(t, dstp, dnx, srcp)
    return _tc_finalize(num, dl)
